# baseline (speedup 1.0000x reference)
import functools

import jax
import jax.numpy as jnp
from jax import lax
from jax.experimental import pallas as pl
from jax.experimental.pallas import tpu as pltpu
from jax.experimental.pallas import tpu_sc as plsc

_N_BINS = 10


def _stage1_body(y_ref, lab_ref, conf_ref, acc_ref):
    yv = y_ref[...]
    bsz, nclass = yv.shape
    m = jnp.max(yv, axis=1, keepdims=True)
    e = jnp.exp(yv - m)
    s = jnp.sum(e, axis=1, keepdims=True)
    idxs = lax.broadcasted_iota(jnp.int32, yv.shape, 1)
    pred = jnp.min(jnp.where(yv == m, idxs, nclass), axis=1, keepdims=True)
    conf_l = jnp.reshape(1.0 / s, (1, bsz))
    pred_l = jnp.reshape(pred, (1, bsz))
    acc_l = (pred_l == lab_ref[0]).astype(jnp.float32)
    conf_ref[0] = conf_l
    acc_ref[0] = acc_l


def _stage1(y, labels, block_rows):
    n, c = y.shape
    nb = n // block_rows
    labels3 = labels.reshape(nb, 1, block_rows)
    conf3, acc3 = pl.pallas_call(
        _stage1_body,
        grid=(nb,),
        in_specs=[
            pl.BlockSpec((block_rows, c), lambda i: (i, 0)),
            pl.BlockSpec((1, 1, block_rows), lambda i: (i, 0, 0)),
        ],
        out_specs=[
            pl.BlockSpec((1, 1, block_rows), lambda i: (i, 0, 0)),
            pl.BlockSpec((1, 1, block_rows), lambda i: (i, 0, 0)),
        ],
        out_shape=[
            jax.ShapeDtypeStruct((nb, 1, block_rows), jnp.float32),
            jax.ShapeDtypeStruct((nb, 1, block_rows), jnp.float32),
        ],
    )(y, labels3)
    return conf3.reshape(-1), acc3.reshape(-1)


def _make_hist(n):
    nw = 32
    per_w = n // nw
    n_vec = per_w // 16
    mesh = plsc.VectorSubcoreMesh(core_axis_name="c", subcore_axis_name="s")

    @functools.partial(
        pl.kernel,
        mesh=mesh,
        out_type=jax.ShapeDtypeStruct((nw, 3 * _N_BINS * 16), jnp.float32),
        scratch_types=[
            pltpu.VMEM((per_w,), jnp.float32),
            pltpu.VMEM((per_w,), jnp.float32),
            pltpu.VMEM((_N_BINS + 1, 16), jnp.float32),
            pltpu.VMEM((3 * _N_BINS * 16,), jnp.float32),
        ],
    )
    def hist(conf_hbm, acc_hbm, bounds_hbm, out_hbm, conf_v, acc_v, bounds_v,
             accum_v):
        wid = lax.axis_index("s") * 2 + lax.axis_index("c")
        base = wid * per_w
        pltpu.sync_copy(conf_hbm.at[pl.ds(base, per_w)], conf_v)
        pltpu.sync_copy(acc_hbm.at[pl.ds(base, per_w)], acc_v)
        pltpu.sync_copy(bounds_hbm, bounds_v)
        zeros16 = jnp.zeros((16,), jnp.float32)
        ones16 = jnp.ones((16,), jnp.float32)
        lo_bounds = [bounds_v[k] for k in range(_N_BINS)]
        hi_bounds = [bounds_v[k] for k in range(1, _N_BINS + 1)]

        def body(i, carry):
            v = conf_v[pl.ds(i * 16, 16)]
            a = acc_v[pl.ds(i * 16, 16)]
            out = []
            for k in range(_N_BINS):
                m = (v > lo_bounds[k]) & (v <= hi_bounds[k])
                cnt = carry[3 * k] + jnp.where(m, ones16, zeros16)
                sconf = carry[3 * k + 1] + jnp.where(m, v, zeros16)
                sacc = carry[3 * k + 2] + jnp.where(m, a, zeros16)
                out += [cnt, sconf, sacc]
            return tuple(out)

        init = tuple(zeros16 for _ in range(3 * _N_BINS))
        final = lax.fori_loop(0, n_vec, body, init)
        for k in range(_N_BINS):
            accum_v[pl.ds(k * 16, 16)] = final[3 * k]
            accum_v[pl.ds((_N_BINS + k) * 16, 16)] = final[3 * k + 1]
            accum_v[pl.ds((2 * _N_BINS + k) * 16, 16)] = final[3 * k + 2]
        pltpu.sync_copy(accum_v, out_hbm.at[wid])

    return hist


def kernel(y, labels):
    n, _ = y.shape
    conf_flat, acc_flat = _stage1(y, labels, block_rows=4096)
    bounds = jnp.linspace(0.0, 1.0, _N_BINS + 1)
    bounds_b = jnp.broadcast_to(bounds[:, None], (_N_BINS + 1, 16))
    partials = _make_hist(n)(conf_flat, acc_flat, bounds_b)
    p = partials.reshape(32, 3, _N_BINS, 16).sum(axis=(0, 3))
    cnt, sconf, sacc = p[0], p[1], p[2]
    denom = jnp.maximum(cnt, 1.0)
    contrib = jnp.abs(sconf / denom - sacc / denom) * (cnt / n)
    ece = jnp.sum(jnp.where(cnt > 0, contrib, 0.0))
    return ece.reshape(1)


# speedup vs baseline: 3.9186x; 3.9186x over previous
import functools

import jax
import jax.numpy as jnp
from jax import lax
from jax.experimental import pallas as pl
from jax.experimental.pallas import tpu as pltpu
from jax.experimental.pallas import tpu_sc as plsc

_N_BINS = 10


def _stage1_body(y_ref, lab_ref, conf_ref, acc_ref):
    yv = y_ref[...]
    _, nclass = yv.shape
    yt = yv.T
    m = jnp.max(yt, axis=0, keepdims=True)
    e = jnp.exp(yt - m)
    ones_row = jnp.ones((1, nclass), jnp.float32)
    s = lax.dot_general(ones_row, e, (((1,), (0,)), ((), ())),
                        preferred_element_type=jnp.float32)
    ismax = (yt == m).astype(jnp.float32)
    iota_row = lax.broadcasted_iota(jnp.int32, (1, nclass), 1).astype(jnp.float32)
    predf = lax.dot_general(iota_row, ismax, (((1,), (0,)), ((), ())),
                            preferred_element_type=jnp.float32)
    conf_ref[0] = 1.0 / s
    lab_f = lab_ref[0].astype(jnp.float32)
    acc_ref[0] = (predf == lab_f).astype(jnp.float32)


def _stage1(y, labels, block_rows):
    n, c = y.shape
    nb = n // block_rows
    labels3 = labels.reshape(nb, 1, block_rows)
    conf3, acc3 = pl.pallas_call(
        _stage1_body,
        grid=(nb,),
        in_specs=[
            pl.BlockSpec((block_rows, c), lambda i: (i, 0)),
            pl.BlockSpec((1, 1, block_rows), lambda i: (i, 0, 0)),
        ],
        out_specs=[
            pl.BlockSpec((1, 1, block_rows), lambda i: (i, 0, 0)),
            pl.BlockSpec((1, 1, block_rows), lambda i: (i, 0, 0)),
        ],
        out_shape=[
            jax.ShapeDtypeStruct((nb, 1, block_rows), jnp.float32),
            jax.ShapeDtypeStruct((nb, 1, block_rows), jnp.float32),
        ],
    )(y, labels3)
    return conf3.reshape(-1), acc3.reshape(-1)


def _make_hist(n):
    nw = 32
    per_w = n // nw
    n_vec = per_w // 16
    mesh = plsc.VectorSubcoreMesh(core_axis_name="c", subcore_axis_name="s")

    @functools.partial(
        pl.kernel,
        mesh=mesh,
        out_type=jax.ShapeDtypeStruct((nw, 3 * _N_BINS * 16), jnp.float32),
        scratch_types=[
            pltpu.VMEM((per_w,), jnp.float32),
            pltpu.VMEM((per_w,), jnp.float32),
            pltpu.VMEM((_N_BINS + 1, 16), jnp.float32),
            pltpu.VMEM((3 * _N_BINS * 16,), jnp.float32),
        ],
    )
    def hist(conf_hbm, acc_hbm, bounds_hbm, out_hbm, conf_v, acc_v, bounds_v,
             accum_v):
        wid = lax.axis_index("s") * 2 + lax.axis_index("c")
        base = wid * per_w
        pltpu.sync_copy(conf_hbm.at[pl.ds(base, per_w)], conf_v)
        pltpu.sync_copy(acc_hbm.at[pl.ds(base, per_w)], acc_v)
        pltpu.sync_copy(bounds_hbm, bounds_v)
        zeros16 = jnp.zeros((16,), jnp.float32)
        ones16 = jnp.ones((16,), jnp.float32)
        lo_bounds = [bounds_v[k] for k in range(_N_BINS)]
        hi_bounds = [bounds_v[k] for k in range(1, _N_BINS + 1)]

        def body(i, carry):
            v = conf_v[pl.ds(i * 16, 16)]
            a = acc_v[pl.ds(i * 16, 16)]
            out = []
            for k in range(_N_BINS):
                m = (v > lo_bounds[k]) & (v <= hi_bounds[k])
                cnt = carry[3 * k] + jnp.where(m, ones16, zeros16)
                sconf = carry[3 * k + 1] + jnp.where(m, v, zeros16)
                sacc = carry[3 * k + 2] + jnp.where(m, a, zeros16)
                out += [cnt, sconf, sacc]
            return tuple(out)

        init = tuple(zeros16 for _ in range(3 * _N_BINS))
        final = lax.fori_loop(0, n_vec, body, init)
        for k in range(_N_BINS):
            accum_v[pl.ds(k * 16, 16)] = final[3 * k]
            accum_v[pl.ds((_N_BINS + k) * 16, 16)] = final[3 * k + 1]
            accum_v[pl.ds((2 * _N_BINS + k) * 16, 16)] = final[3 * k + 2]
        pltpu.sync_copy(accum_v, out_hbm.at[wid])

    return hist


def kernel(y, labels):
    n, _ = y.shape
    conf_flat, acc_flat = _stage1(y, labels, block_rows=4096)
    bounds = jnp.linspace(0.0, 1.0, _N_BINS + 1)
    bounds_b = jnp.broadcast_to(bounds[:, None], (_N_BINS + 1, 16))
    partials = _make_hist(n)(conf_flat, acc_flat, bounds_b)
    p = partials.reshape(32, 3, _N_BINS, 16).sum(axis=(0, 3))
    cnt, sconf, sacc = p[0], p[1], p[2]
    denom = jnp.maximum(cnt, 1.0)
    contrib = jnp.abs(sconf / denom - sacc / denom) * (cnt / n)
    ece = jnp.sum(jnp.where(cnt > 0, contrib, 0.0))
    return ece.reshape(1)


# speedup vs baseline: 5.0622x; 1.2919x over previous
import functools

import jax
import jax.numpy as jnp
from jax import lax
from jax.experimental import pallas as pl
from jax.experimental.pallas import tpu as pltpu
from jax.experimental.pallas import tpu_sc as plsc

_N_BINS = 10


def _stage1_body(y_ref, lab_ref, conf_ref, acc_ref):
    yv = y_ref[...]
    _, nclass = yv.shape
    yt = yv.T
    m = jnp.max(yt, axis=0, keepdims=True)
    e = jnp.exp(yt - m)
    ones_row = jnp.ones((1, nclass), jnp.float32)
    s = lax.dot_general(ones_row, e, (((1,), (0,)), ((), ())),
                        preferred_element_type=jnp.float32)
    ismax = (yt == m).astype(jnp.float32)
    iota_row = lax.broadcasted_iota(jnp.int32, (1, nclass), 1).astype(jnp.float32)
    predf = lax.dot_general(iota_row, ismax, (((1,), (0,)), ((), ())),
                            preferred_element_type=jnp.float32)
    conf_ref[0] = 1.0 / s
    lab_f = lab_ref[0].astype(jnp.float32)
    acc_ref[0] = (predf == lab_f).astype(jnp.float32)


def _stage1(y, labels, block_rows):
    n, c = y.shape
    nb = n // block_rows
    labels3 = labels.reshape(nb, 1, block_rows)
    conf3, acc3 = pl.pallas_call(
        _stage1_body,
        grid=(nb,),
        in_specs=[
            pl.BlockSpec((block_rows, c), lambda i: (i, 0)),
            pl.BlockSpec((1, 1, block_rows), lambda i: (i, 0, 0)),
        ],
        out_specs=[
            pl.BlockSpec((1, 1, block_rows), lambda i: (i, 0, 0)),
            pl.BlockSpec((1, 1, block_rows), lambda i: (i, 0, 0)),
        ],
        out_shape=[
            jax.ShapeDtypeStruct((nb, 1, block_rows), jnp.float32),
            jax.ShapeDtypeStruct((nb, 1, block_rows), jnp.float32),
        ],
    )(y, labels3)
    return conf3.reshape(-1), acc3.reshape(-1)


def _make_hist(n):
    nw = 32
    per_w = n // nw
    n_vec = per_w // 16
    mesh = plsc.VectorSubcoreMesh(core_axis_name="c", subcore_axis_name="s")

    @functools.partial(
        pl.kernel,
        mesh=mesh,
        out_type=jax.ShapeDtypeStruct((nw, (2 + 3 * (_N_BINS - 1)) * 16),
                                      jnp.float32),
        scratch_types=[
            pltpu.VMEM((per_w,), jnp.float32),
            pltpu.VMEM((per_w,), jnp.float32),
            pltpu.VMEM((_N_BINS + 1, 16), jnp.float32),
            pltpu.VMEM(((2 + 3 * (_N_BINS - 1)) * 16,), jnp.float32),
        ],
    )
    def hist(conf_hbm, acc_hbm, bounds_hbm, out_hbm, conf_v, acc_v, bounds_v,
             accum_v):
        wid = lax.axis_index("s") * 2 + lax.axis_index("c")
        base = wid * per_w
        pltpu.sync_copy(conf_hbm.at[pl.ds(base, per_w)], conf_v)
        pltpu.sync_copy(acc_hbm.at[pl.ds(base, per_w)], acc_v)
        pltpu.sync_copy(bounds_hbm, bounds_v)
        zeros16 = jnp.zeros((16,), jnp.float32)
        ones16 = jnp.ones((16,), jnp.float32)
        thr = [bounds_v[k] for k in range(1, _N_BINS)]

        def body(i, carry):
            v = conf_v[pl.ds(i * 16, 16)]
            a = acc_v[pl.ds(i * 16, 16)]
            out = [carry[0] + v, carry[1] + a]
            for k in range(_N_BINS - 1):
                m = v > thr[k]
                out.append(carry[3 * k + 2] + jnp.where(m, ones16, zeros16))
                out.append(carry[3 * k + 3] + jnp.where(m, v, zeros16))
                out.append(carry[3 * k + 4] + jnp.where(m, a, zeros16))
            return tuple(out)

        init = tuple(zeros16 for _ in range(2 + 3 * (_N_BINS - 1)))
        final = lax.fori_loop(0, n_vec, body, init)
        for j in range(2 + 3 * (_N_BINS - 1)):
            accum_v[pl.ds(j * 16, 16)] = final[j]
        pltpu.sync_copy(accum_v, out_hbm.at[wid])

    return hist


def kernel(y, labels):
    n, _ = y.shape
    conf_flat, acc_flat = _stage1(y, labels, block_rows=8192)
    bounds = jnp.linspace(0.0, 1.0, _N_BINS + 1)
    bounds_b = jnp.broadcast_to(bounds[:, None], (_N_BINS + 1, 16))
    partials = _make_hist(n)(conf_flat, acc_flat, bounds_b)
    p = partials.reshape(32, 2 + 3 * (_N_BINS - 1), 16).sum(axis=(0, 2))
    tot_v, tot_a = p[0], p[1]
    exc = p[2:].reshape(_N_BINS - 1, 3)
    c_exc = jnp.concatenate([jnp.array([float(n)]), exc[:, 0],
                             jnp.array([0.0])])
    s_exc = jnp.concatenate([tot_v[None], exc[:, 1], jnp.array([0.0])])
    a_exc = jnp.concatenate([tot_a[None], exc[:, 2], jnp.array([0.0])])
    cnt = c_exc[:-1] - c_exc[1:]
    sconf = s_exc[:-1] - s_exc[1:]
    sacc = a_exc[:-1] - a_exc[1:]
    denom = jnp.maximum(cnt, 1.0)
    contrib = jnp.abs(sconf / denom - sacc / denom) * (cnt / n)
    ece = jnp.sum(jnp.where(cnt > 0, contrib, 0.0))
    return ece.reshape(1)


# speedup vs baseline: 5.2234x; 1.0318x over previous
import functools

import jax
import jax.numpy as jnp
from jax import lax
from jax.experimental import pallas as pl
from jax.experimental.pallas import tpu as pltpu
from jax.experimental.pallas import tpu_sc as plsc

_N_BINS = 10


def _stage1_body(y_ref, lab_ref, conf_ref, acc_ref):
    yv = y_ref[...]
    _, nclass = yv.shape
    yt = yv.T
    m = jnp.max(yt, axis=0, keepdims=True)
    e = jnp.exp(yt - m)
    ones_row = jnp.ones((1, nclass), jnp.float32)
    s = lax.dot_general(ones_row, e, (((1,), (0,)), ((), ())),
                        preferred_element_type=jnp.float32)
    ismax = (yt == m).astype(jnp.float32)
    iota_row = lax.broadcasted_iota(jnp.int32, (1, nclass), 1).astype(jnp.float32)
    predf = lax.dot_general(iota_row, ismax, (((1,), (0,)), ((), ())),
                            preferred_element_type=jnp.float32)
    conf_ref[0] = 1.0 / s
    lab_f = lab_ref[0].astype(jnp.float32)
    acc_ref[0] = (predf == lab_f).astype(jnp.float32)


def _stage1(y, labels3, block_rows, block_off, nb):
    _, c = y.shape
    conf3, acc3 = pl.pallas_call(
        _stage1_body,
        grid=(nb,),
        in_specs=[
            pl.BlockSpec((block_rows, c), lambda i: (i + block_off, 0)),
            pl.BlockSpec((1, 1, block_rows), lambda i: (i + block_off, 0, 0)),
        ],
        out_specs=[
            pl.BlockSpec((1, 1, block_rows), lambda i: (i, 0, 0)),
            pl.BlockSpec((1, 1, block_rows), lambda i: (i, 0, 0)),
        ],
        out_shape=[
            jax.ShapeDtypeStruct((nb, 1, block_rows), jnp.float32),
            jax.ShapeDtypeStruct((nb, 1, block_rows), jnp.float32),
        ],
    )(y, labels3)
    return conf3.reshape(-1), acc3.reshape(-1)


def _make_hist(n):
    nw = 32
    per_w = n // nw
    n_vec = per_w // 16
    mesh = plsc.VectorSubcoreMesh(core_axis_name="c", subcore_axis_name="s")

    @functools.partial(
        pl.kernel,
        mesh=mesh,
        out_type=jax.ShapeDtypeStruct((nw, (2 + 3 * (_N_BINS - 1)) * 16),
                                      jnp.float32),
        scratch_types=[
            pltpu.VMEM((per_w,), jnp.float32),
            pltpu.VMEM((per_w,), jnp.float32),
            pltpu.VMEM((_N_BINS + 1, 16), jnp.float32),
            pltpu.VMEM(((2 + 3 * (_N_BINS - 1)) * 16,), jnp.float32),
        ],
    )
    def hist(conf_hbm, acc_hbm, bounds_hbm, out_hbm, conf_v, acc_v, bounds_v,
             accum_v):
        wid = lax.axis_index("s") * 2 + lax.axis_index("c")
        base = wid * per_w
        pltpu.sync_copy(conf_hbm.at[pl.ds(base, per_w)], conf_v)
        pltpu.sync_copy(acc_hbm.at[pl.ds(base, per_w)], acc_v)
        pltpu.sync_copy(bounds_hbm, bounds_v)
        zeros16 = jnp.zeros((16,), jnp.float32)
        ones16 = jnp.ones((16,), jnp.float32)
        thr = [bounds_v[k] for k in range(1, _N_BINS)]

        def body(i, carry):
            v = conf_v[pl.ds(i * 16, 16)]
            a = acc_v[pl.ds(i * 16, 16)]
            out = [carry[0] + v, carry[1] + a]
            for k in range(_N_BINS - 1):
                m = v > thr[k]
                out.append(carry[3 * k + 2] + jnp.where(m, ones16, zeros16))
                out.append(carry[3 * k + 3] + jnp.where(m, v, zeros16))
                out.append(carry[3 * k + 4] + jnp.where(m, a, zeros16))
            return tuple(out)

        init = tuple(zeros16 for _ in range(2 + 3 * (_N_BINS - 1)))
        final = lax.fori_loop(0, n_vec, body, init)
        for j in range(2 + 3 * (_N_BINS - 1)):
            accum_v[pl.ds(j * 16, 16)] = final[j]
        pltpu.sync_copy(accum_v, out_hbm.at[wid])

    return hist


def kernel(y, labels):
    n, _ = y.shape
    block_rows = 8192
    n_chunks = 2
    nb_total = n // block_rows
    nb = nb_total // n_chunks
    labels3 = labels.reshape(nb_total, 1, block_rows)
    bounds = jnp.linspace(0.0, 1.0, _N_BINS + 1)
    bounds_b = jnp.broadcast_to(bounds[:, None], (_N_BINS + 1, 16))
    hist = _make_hist(n // n_chunks)
    partial_list = []
    for ci in range(n_chunks):
        conf_flat, acc_flat = _stage1(y, labels3, block_rows, ci * nb, nb)
        partial_list.append(hist(conf_flat, acc_flat, bounds_b))
    partials = sum(partial_list)
    p = partials.reshape(32, 2 + 3 * (_N_BINS - 1), 16).sum(axis=(0, 2))
    tot_v, tot_a = p[0], p[1]
    exc = p[2:].reshape(_N_BINS - 1, 3)
    c_exc = jnp.concatenate([jnp.array([float(n)]), exc[:, 0],
                             jnp.array([0.0])])
    s_exc = jnp.concatenate([tot_v[None], exc[:, 1], jnp.array([0.0])])
    a_exc = jnp.concatenate([tot_a[None], exc[:, 2], jnp.array([0.0])])
    cnt = c_exc[:-1] - c_exc[1:]
    sconf = s_exc[:-1] - s_exc[1:]
    sacc = a_exc[:-1] - a_exc[1:]
    denom = jnp.maximum(cnt, 1.0)
    contrib = jnp.abs(sconf / denom - sacc / denom) * (cnt / n)
    ece = jnp.sum(jnp.where(cnt > 0, contrib, 0.0))
    return ece.reshape(1)


# speedup vs baseline: 5.9432x; 1.1378x over previous
import functools

import jax
import jax.numpy as jnp
from jax import lax
from jax.experimental import pallas as pl
from jax.experimental.pallas import tpu as pltpu
from jax.experimental.pallas import tpu_sc as plsc

_N_BINS = 10


def _stage1_body(y_ref, lab_ref, conf_ref, acc_ref):
    yv = y_ref[...]
    _, nclass = yv.shape
    yt = yv.T
    m = jnp.max(yt, axis=0, keepdims=True)
    e = jnp.exp(yt - m)
    ones_row = jnp.ones((1, nclass), jnp.float32)
    s = lax.dot_general(ones_row, e, (((1,), (0,)), ((), ())),
                        preferred_element_type=jnp.float32)
    ismax = (yt == m).astype(jnp.float32)
    iota_row = lax.broadcasted_iota(jnp.int32, (1, nclass), 1).astype(jnp.float32)
    predf = lax.dot_general(iota_row, ismax, (((1,), (0,)), ((), ())),
                            preferred_element_type=jnp.float32)
    conf_ref[0] = 1.0 / s
    lab_f = lab_ref[0].astype(jnp.float32)
    acc_ref[0] = (predf == lab_f).astype(jnp.float32)


def _stage1(y, labels3, block_rows, block_off, nb):
    _, c = y.shape
    conf3, acc3 = pl.pallas_call(
        _stage1_body,
        grid=(nb,),
        in_specs=[
            pl.BlockSpec((block_rows, c), lambda i: (i + block_off, 0)),
            pl.BlockSpec((1, 1, block_rows), lambda i: (i + block_off, 0, 0)),
        ],
        out_specs=[
            pl.BlockSpec((1, 1, block_rows), lambda i: (i, 0, 0)),
            pl.BlockSpec((1, 1, block_rows), lambda i: (i, 0, 0)),
        ],
        out_shape=[
            jax.ShapeDtypeStruct((nb, 1, block_rows), jnp.float32),
            jax.ShapeDtypeStruct((nb, 1, block_rows), jnp.float32),
        ],
    )(y, labels3)
    return conf3.reshape(-1), acc3.reshape(-1)


def _make_hist(n):
    nw = 32
    per_w = n // nw
    n_vec = per_w // 16
    mesh = plsc.VectorSubcoreMesh(core_axis_name="c", subcore_axis_name="s")

    @functools.partial(
        pl.kernel,
        mesh=mesh,
        out_type=jax.ShapeDtypeStruct((nw, (2 + 3 * (_N_BINS - 1)) * 16),
                                      jnp.float32),
        scratch_types=[
            pltpu.VMEM((per_w,), jnp.float32),
            pltpu.VMEM((per_w,), jnp.float32),
            pltpu.VMEM((_N_BINS + 1, 16), jnp.float32),
            pltpu.VMEM(((2 + 3 * (_N_BINS - 1)) * 16,), jnp.float32),
        ],
    )
    def hist(conf_hbm, acc_hbm, bounds_hbm, out_hbm, conf_v, acc_v, bounds_v,
             accum_v):
        wid = lax.axis_index("s") * 2 + lax.axis_index("c")
        base = wid * per_w
        pltpu.sync_copy(conf_hbm.at[pl.ds(base, per_w)], conf_v)
        pltpu.sync_copy(acc_hbm.at[pl.ds(base, per_w)], acc_v)
        pltpu.sync_copy(bounds_hbm, bounds_v)
        zeros16 = jnp.zeros((16,), jnp.float32)
        ones16 = jnp.ones((16,), jnp.float32)
        thr = [bounds_v[k] for k in range(1, _N_BINS)]

        def body(i, carry):
            v = conf_v[pl.ds(i * 16, 16)]
            a = acc_v[pl.ds(i * 16, 16)]
            out = [carry[0] + v, carry[1] + a]
            for k in range(_N_BINS - 1):
                m = v > thr[k]
                out.append(carry[3 * k + 2] + jnp.where(m, ones16, zeros16))
                out.append(carry[3 * k + 3] + jnp.where(m, v, zeros16))
                out.append(carry[3 * k + 4] + jnp.where(m, a, zeros16))
            return tuple(out)

        init = tuple(zeros16 for _ in range(2 + 3 * (_N_BINS - 1)))
        final = lax.fori_loop(0, n_vec, body, init)
        for j in range(2 + 3 * (_N_BINS - 1)):
            accum_v[pl.ds(j * 16, 16)] = final[j]
        pltpu.sync_copy(accum_v, out_hbm.at[wid])

    return hist


def kernel(y, labels):
    n, _ = y.shape
    block_rows = 16384
    n_chunks = 2
    nb_total = n // block_rows
    nb = nb_total // n_chunks
    labels3 = labels.reshape(nb_total, 1, block_rows)
    bounds = jnp.linspace(0.0, 1.0, _N_BINS + 1)
    bounds_b = jnp.broadcast_to(bounds[:, None], (_N_BINS + 1, 16))
    hist = _make_hist(n // n_chunks)
    partial_list = []
    for ci in range(n_chunks):
        conf_flat, acc_flat = _stage1(y, labels3, block_rows, ci * nb, nb)
        partial_list.append(hist(conf_flat, acc_flat, bounds_b))
    partials = sum(partial_list)
    p = partials.reshape(32, 2 + 3 * (_N_BINS - 1), 16).sum(axis=(0, 2))
    tot_v, tot_a = p[0], p[1]
    exc = p[2:].reshape(_N_BINS - 1, 3)
    c_exc = jnp.concatenate([jnp.array([float(n)]), exc[:, 0],
                             jnp.array([0.0])])
    s_exc = jnp.concatenate([tot_v[None], exc[:, 1], jnp.array([0.0])])
    a_exc = jnp.concatenate([tot_a[None], exc[:, 2], jnp.array([0.0])])
    cnt = c_exc[:-1] - c_exc[1:]
    sconf = s_exc[:-1] - s_exc[1:]
    sacc = a_exc[:-1] - a_exc[1:]
    denom = jnp.maximum(cnt, 1.0)
    contrib = jnp.abs(sconf / denom - sacc / denom) * (cnt / n)
    ece = jnp.sum(jnp.where(cnt > 0, contrib, 0.0))
    return ece.reshape(1)


# speedup vs baseline: 6.3396x; 1.0667x over previous
import functools

import jax
import jax.numpy as jnp
from jax import lax
from jax.experimental import pallas as pl
from jax.experimental.pallas import tpu as pltpu
from jax.experimental.pallas import tpu_sc as plsc

_N_BINS = 10


def _stage1_body(y_ref, lab_ref, conf_ref, acc_ref):
    yv = y_ref[...]
    _, nclass = yv.shape
    yt = yv.T
    m = jnp.max(yt, axis=0, keepdims=True)
    e = jnp.exp(yt - m)
    ones_row = jnp.ones((1, nclass), jnp.float32)
    s = lax.dot_general(ones_row, e, (((1,), (0,)), ((), ())),
                        preferred_element_type=jnp.float32)
    ismax = (yt == m).astype(jnp.float32)
    iota_row = lax.broadcasted_iota(jnp.int32, (1, nclass), 1).astype(jnp.float32)
    predf = lax.dot_general(iota_row, ismax, (((1,), (0,)), ((), ())),
                            preferred_element_type=jnp.float32)
    conf_ref[0] = 1.0 / s
    lab_f = lab_ref[0].astype(jnp.float32)
    acc_ref[0] = (predf == lab_f).astype(jnp.float32)


def _stage1(y, labels3, block_rows, block_off, nb):
    _, c = y.shape
    conf3, acc3 = pl.pallas_call(
        _stage1_body,
        grid=(nb,),
        in_specs=[
            pl.BlockSpec((block_rows, c), lambda i: (i + block_off, 0)),
            pl.BlockSpec((1, 1, block_rows), lambda i: (i + block_off, 0, 0)),
        ],
        out_specs=[
            pl.BlockSpec((1, 1, block_rows), lambda i: (i, 0, 0)),
            pl.BlockSpec((1, 1, block_rows), lambda i: (i, 0, 0)),
        ],
        out_shape=[
            jax.ShapeDtypeStruct((nb, 1, block_rows), jnp.float32),
            jax.ShapeDtypeStruct((nb, 1, block_rows), jnp.float32),
        ],
    )(y, labels3)
    return conf3.reshape(-1), acc3.reshape(-1)


def _make_hist(n):
    nw = 32
    per_w = n // nw
    n_vec = per_w // 16
    mesh = plsc.VectorSubcoreMesh(core_axis_name="c", subcore_axis_name="s")

    @functools.partial(
        pl.kernel,
        mesh=mesh,
        out_type=jax.ShapeDtypeStruct((nw, (2 + 3 * (_N_BINS - 1)) * 16),
                                      jnp.float32),
        scratch_types=[
            pltpu.VMEM((per_w,), jnp.float32),
            pltpu.VMEM((per_w,), jnp.float32),
            pltpu.VMEM((_N_BINS + 1, 16), jnp.float32),
            pltpu.VMEM(((2 + 3 * (_N_BINS - 1)) * 16,), jnp.float32),
        ],
    )
    def hist(conf_hbm, acc_hbm, bounds_hbm, out_hbm, conf_v, acc_v, bounds_v,
             accum_v):
        wid = lax.axis_index("s") * 2 + lax.axis_index("c")
        base = wid * per_w
        pltpu.sync_copy(conf_hbm.at[pl.ds(base, per_w)], conf_v)
        pltpu.sync_copy(acc_hbm.at[pl.ds(base, per_w)], acc_v)
        pltpu.sync_copy(bounds_hbm, bounds_v)
        zeros16 = jnp.zeros((16,), jnp.float32)
        ones16 = jnp.ones((16,), jnp.float32)
        thr = [bounds_v[k] for k in range(1, _N_BINS)]

        def body(i, carry):
            v = conf_v[pl.ds(i * 16, 16)]
            a = acc_v[pl.ds(i * 16, 16)]
            out = [carry[0] + v, carry[1] + a]
            for k in range(_N_BINS - 1):
                m = v > thr[k]
                out.append(carry[3 * k + 2] + jnp.where(m, ones16, zeros16))
                out.append(carry[3 * k + 3] + jnp.where(m, v, zeros16))
                out.append(carry[3 * k + 4] + jnp.where(m, a, zeros16))
            return tuple(out)

        init = tuple(zeros16 for _ in range(2 + 3 * (_N_BINS - 1)))
        final = lax.fori_loop(0, n_vec, body, init)
        for j in range(2 + 3 * (_N_BINS - 1)):
            accum_v[pl.ds(j * 16, 16)] = final[j]
        pltpu.sync_copy(accum_v, out_hbm.at[wid])

    return hist


def kernel(y, labels):
    n, _ = y.shape
    block_rows = 32768
    n_chunks = 2
    nb_total = n // block_rows
    nb = nb_total // n_chunks
    labels3 = labels.reshape(nb_total, 1, block_rows)
    bounds = jnp.linspace(0.0, 1.0, _N_BINS + 1)
    bounds_b = jnp.broadcast_to(bounds[:, None], (_N_BINS + 1, 16))
    hist = _make_hist(n // n_chunks)
    partial_list = []
    for ci in range(n_chunks):
        conf_flat, acc_flat = _stage1(y, labels3, block_rows, ci * nb, nb)
        partial_list.append(hist(conf_flat, acc_flat, bounds_b))
    partials = sum(partial_list)
    p = partials.reshape(32, 2 + 3 * (_N_BINS - 1), 16).sum(axis=(0, 2))
    tot_v, tot_a = p[0], p[1]
    exc = p[2:].reshape(_N_BINS - 1, 3)
    c_exc = jnp.concatenate([jnp.array([float(n)]), exc[:, 0],
                             jnp.array([0.0])])
    s_exc = jnp.concatenate([tot_v[None], exc[:, 1], jnp.array([0.0])])
    a_exc = jnp.concatenate([tot_a[None], exc[:, 2], jnp.array([0.0])])
    cnt = c_exc[:-1] - c_exc[1:]
    sconf = s_exc[:-1] - s_exc[1:]
    sacc = a_exc[:-1] - a_exc[1:]
    denom = jnp.maximum(cnt, 1.0)
    contrib = jnp.abs(sconf / denom - sacc / denom) * (cnt / n)
    ece = jnp.sum(jnp.where(cnt > 0, contrib, 0.0))
    return ece.reshape(1)
